# baseline (speedup 1.0000x reference)
import functools

import jax
import jax.numpy as jnp
from jax.experimental import pallas as pl
from jax.experimental.pallas import tpu as pltpu
from jax.experimental.pallas import tpu_sc as plsc

N = 4096
DIM = 1024
E = 8
K = 2
UD = 16
FF = 4 * DIM

T = 512
MAX_TILES = 24
P = MAX_TILES * T
FT = 4096
JF = FF // FT

_SQRT_2 = 1.4142135623730951


def _erf(x):
    a1, a2, a3, a4, a5 = (
        0.254829592, -0.284496736, 1.421413741, -1.453152027, 1.061405429)
    p = 0.3275911
    s = jnp.sign(x)
    ax = jnp.abs(x)
    t = 1.0 / (1.0 + p * ax)
    poly = ((((a5 * t + a4) * t + a3) * t + a2) * t + a1) * t
    y = 1.0 - poly * jnp.exp(-ax * ax)
    return s * y


def _gelu_exact(x):
    return x * 0.5 * (1.0 + _erf(x / _SQRT_2))


def _router_body(x_ref, wu1_ref, bu1_ref, wu2_ref, bu2_ref, wx_ref, wru_ref,
                 w0_ref, w1_ref, d0_ref, d1_ref, te_ref, tv_ref, aux_ref):
    x = x_ref[...]
    h = _gelu_exact(
        jnp.dot(x, wu1_ref[...], preferred_element_type=jnp.float32)
        + bu1_ref[...])
    z = jnp.dot(h, wu2_ref[...], preferred_element_type=jnp.float32) \
        + bu2_ref[...]
    u = jnp.maximum(z, 0.0) + jnp.log(1.0 + jnp.exp(-jnp.abs(z)))
    un = u / (jnp.mean(u) + 1e-8)

    logits = (jnp.dot(x, wx_ref[...], preferred_element_type=jnp.float32)
              + un * wru_ref[...])
    m = jnp.max(logits, axis=-1, keepdims=True)
    ex = jnp.exp(logits - m)
    ssum = jnp.sum(ex, axis=-1, keepdims=True)
    probs = ex / ssum

    neg = jnp.full((N, 1), -1.0, jnp.float32)
    b0 = neg
    b1 = neg
    i0 = jnp.zeros((N, 1), jnp.int32)
    i1 = jnp.zeros((N, 1), jnp.int32)
    for e in range(E):
        pe = probs[:, e:e + 1]
        gt0 = pe > b0
        gt1 = pe > b1
        nb1 = jnp.where(gt0, b0, jnp.where(gt1, pe, b1))
        ni1 = jnp.where(gt0, i0, jnp.where(gt1, e, i1))
        b0 = jnp.where(gt0, pe, b0)
        i0 = jnp.where(gt0, e, i0)
        b1 = nb1
        i1 = ni1
    wsum = b0 + b1
    w0_ref[...] = b0 / wsum
    w1_ref[...] = b1 / wsum

    lanes = jax.lax.broadcasted_iota(jnp.int32, (N, E), 1)
    oh0 = (i0 == lanes).astype(jnp.float32)
    oh1 = (i1 == lanes).astype(jnp.float32)
    c0 = oh0
    c1 = oh1
    s = 1
    while s < N:
        zpad = jnp.zeros((s, E), jnp.float32)
        c0 = c0 + jnp.concatenate([zpad, c0[:N - s]], axis=0)
        c1 = c1 + jnp.concatenate([zpad, c1[:N - s]], axis=0)
        s *= 2
    cnt0 = c0[N - 1:N, :]
    cnt1 = c1[N - 1:N, :]
    cnt = cnt0 + cnt1
    ps = jnp.ceil(cnt / float(T)) * float(T)
    ee = jax.lax.broadcasted_iota(jnp.int32, (E, E), 0)
    jj = jax.lax.broadcasted_iota(jnp.int32, (E, E), 1)
    off = jnp.sum(jnp.where(jj < ee, ps, 0.0), axis=1, keepdims=True)
    off_b = off.reshape(1, E)

    d0 = jnp.sum(oh0 * (off_b + c0 - 1.0), axis=1, keepdims=True)
    d1 = jnp.sum(oh1 * (off_b + cnt0 + c1 - 1.0), axis=1, keepdims=True)
    d0_ref[...] = d0.astype(jnp.int32)
    d1_ref[...] = d1.astype(jnp.int32)

    total = jnp.sum(ps)
    ends = off_b + ps
    tstart = jax.lax.broadcasted_iota(
        jnp.int32, (64, 1), 0).astype(jnp.float32) * float(T)
    te = jnp.sum((jnp.broadcast_to(ends, (64, E)) <= tstart).astype(jnp.int32),
                 axis=1, keepdims=True)
    te_ref[...] = jnp.minimum(te, E - 1)
    tv_ref[...] = (tstart < total).astype(jnp.int32)

    usage = jnp.mean(probs, axis=0, keepdims=True)
    sel = cnt / float(N * K)
    lb = float(E) * jnp.sum(usage * sel)
    lse = m + jnp.log(ssum)
    zl = jnp.mean(lse * lse)
    aux_ref[...] = (0.01 * lb + 0.01 * zl).reshape(1, 1)


def _ffn_body(te_ref, tv_ref, xs_ref, ws_ref, w1_ref, b1_ref, w2_ref, b2_ref,
              ys_ref):
    i = pl.program_id(0)
    j = pl.program_id(1)

    @pl.when(tv_ref[i] == 1)
    def _():
        h = (jnp.dot(xs_ref[...].astype(jnp.bfloat16), w1_ref[0],
                     preferred_element_type=jnp.float32) + b1_ref[0])
        g = _gelu_exact(h)
        contrib = jnp.dot(g.astype(jnp.bfloat16), w2_ref[0],
                          preferred_element_type=jnp.float32)
        if JF == 1:
            ys_ref[...] = (contrib + b2_ref[0]) * ws_ref[:, 0:1]
        else:
            @pl.when(j == 0)
            def _():
                ys_ref[...] = contrib

            @pl.when(jnp.logical_and(j > 0, j < JF - 1))
            def _():
                ys_ref[...] += contrib

            @pl.when(j == JF - 1)
            def _():
                ys_ref[...] = (ys_ref[...] + contrib + b2_ref[0]) \
                    * ws_ref[:, 0:1]


_NC = 2
_NS = 16
_NW = _NC * _NS
_CH = 64
_CH2 = 32


def _dispatch_call(x, w2n, d0, d1):
    mesh = plsc.VectorSubcoreMesh(core_axis_name="c", subcore_axis_name="s")

    @functools.partial(
        pl.kernel, mesh=mesh,
        out_type=(jax.ShapeDtypeStruct((P, DIM), jnp.float32),
                  jax.ShapeDtypeStruct((P, 128), jnp.float32)),
        scratch_types=[
            pltpu.VMEM((_CH, DIM), jnp.float32),
            pltpu.VMEM((_CH, 128), jnp.float32),
            pltpu.VMEM((_CH, 128), jnp.float32),
            pltpu.VMEM((_CH,), jnp.int32),
            pltpu.VMEM((_CH,), jnp.int32),
            pltpu.SemaphoreType.DMA,
            pltpu.SemaphoreType.DMA,
        ])
    def k(x_hbm, w_hbm, d0_hbm, d1_hbm, xs_hbm, ws_hbm,
          rows_v, wa_v, wb_v, i0_v, i1_v, lsem, ssem):
        wid = jax.lax.axis_index("s") * _NC + jax.lax.axis_index("c")
        per_w = N // _NW

        @pl.loop(0, per_w, step=_CH)
        def _(off):
            base = wid * per_w + off
            cps = [
                pltpu.make_async_copy(x_hbm.at[pl.ds(base, _CH)], rows_v,
                                      lsem),
                pltpu.make_async_copy(w_hbm.at[pl.ds(base, _CH)], wa_v, lsem),
                pltpu.make_async_copy(w_hbm.at[pl.ds(N + base, _CH)], wb_v,
                                      lsem),
                pltpu.make_async_copy(d0_hbm.at[pl.ds(base, _CH)], i0_v,
                                      lsem),
                pltpu.make_async_copy(d1_hbm.at[pl.ds(base, _CH)], i1_v,
                                      lsem),
            ]
            for cp in cps:
                cp.start()
            for cp in cps:
                cp.wait()
            scs = [
                pltpu.make_async_copy(rows_v, xs_hbm.at[i0_v], ssem),
                pltpu.make_async_copy(rows_v, xs_hbm.at[i1_v], ssem),
                pltpu.make_async_copy(wa_v, ws_hbm.at[i0_v], ssem),
                pltpu.make_async_copy(wb_v, ws_hbm.at[i1_v], ssem),
            ]
            for cp in scs:
                cp.start()
            for cp in scs:
                cp.wait()

    return k(x, w2n, d0, d1)


def _combine_call(ys, d0, d1):
    mesh = plsc.VectorSubcoreMesh(core_axis_name="c", subcore_axis_name="s")

    @functools.partial(
        pl.kernel, mesh=mesh,
        out_type=jax.ShapeDtypeStruct((N, DIM), jnp.float32),
        scratch_types=[
            pltpu.VMEM((_CH2, DIM), jnp.float32),
            pltpu.VMEM((_CH2, DIM), jnp.float32),
            pltpu.VMEM((_CH2,), jnp.int32),
            pltpu.VMEM((_CH2,), jnp.int32),
            pltpu.SemaphoreType.DMA,
        ])
    def k(ys_hbm, d0_hbm, d1_hbm, o_hbm, g0_v, g1_v, i0_v, i1_v, sem):
        wid = jax.lax.axis_index("s") * _NC + jax.lax.axis_index("c")
        per_w = N // _NW

        @pl.loop(0, per_w, step=_CH2)
        def _(off):
            base = wid * per_w + off
            pltpu.sync_copy(d0_hbm.at[pl.ds(base, _CH2)], i0_v)
            pltpu.sync_copy(d1_hbm.at[pl.ds(base, _CH2)], i1_v)
            cps = [
                pltpu.make_async_copy(ys_hbm.at[i0_v], g0_v, sem),
                pltpu.make_async_copy(ys_hbm.at[i1_v], g1_v, sem),
            ]
            for cp in cps:
                cp.start()
            for cp in cps:
                cp.wait()

            @pl.loop(0, _CH2)
            def _(r):
                @pl.loop(0, DIM, step=64)
                def _(c):
                    for u in range(4):
                        sl = (r, pl.ds(c + 16 * u, 16))
                        g0_v[sl] = g0_v[sl] + g1_v[sl]

            pltpu.sync_copy(g0_v, o_hbm.at[pl.ds(base, _CH2)])

    return k(ys, d0, d1)


def _cast_body(x_ref, o_ref):
    o_ref[...] = x_ref[...].astype(jnp.bfloat16)


def _cast_call(w):
    e, a, b = w.shape
    cb = 1024
    return pl.pallas_call(
        _cast_body,
        grid=(e, b // cb),
        in_specs=[pl.BlockSpec((1, a, cb), lambda i, j: (i, 0, j))],
        out_specs=pl.BlockSpec((1, a, cb), lambda i, j: (i, 0, j)),
        out_shape=jax.ShapeDtypeStruct((e, a, b), jnp.bfloat16),
    )(w)


def _router_call(x, Wu1, bu1, Wu2, bu2, Wx, Wru):
    outs = (
        jax.ShapeDtypeStruct((N, 1), jnp.float32),
        jax.ShapeDtypeStruct((N, 1), jnp.float32),
        jax.ShapeDtypeStruct((N, 1), jnp.int32),
        jax.ShapeDtypeStruct((N, 1), jnp.int32),
        jax.ShapeDtypeStruct((64, 1), jnp.int32),
        jax.ShapeDtypeStruct((64, 1), jnp.int32),
        jax.ShapeDtypeStruct((1, 1), jnp.float32),
    )
    return pl.pallas_call(_router_body, out_shape=outs)(
        x, Wu1, bu1.reshape(1, UD), Wu2, bu2.reshape(1, 1), Wx,
        Wru.reshape(1, E))


def _ffn_call(te, tv, xs, ws, We1, be1, We2, be2):
    grid_spec = pltpu.PrefetchScalarGridSpec(
        num_scalar_prefetch=2,
        grid=(MAX_TILES, JF),
        in_specs=[
            pl.BlockSpec((T, DIM), lambda i, j, te, tv: (i, 0)),
            pl.BlockSpec((T, 128), lambda i, j, te, tv: (i, 0)),
            pl.BlockSpec((1, DIM, FT), lambda i, j, te, tv: (te[i], 0, j)),
            pl.BlockSpec((1, 1, FT), lambda i, j, te, tv: (te[i], 0, j)),
            pl.BlockSpec((1, FT, DIM), lambda i, j, te, tv: (te[i], j, 0)),
            pl.BlockSpec((1, 1, DIM), lambda i, j, te, tv: (te[i], 0, 0)),
        ],
        out_specs=pl.BlockSpec((T, DIM), lambda i, j, te, tv: (i, 0)),
    )
    return pl.pallas_call(
        _ffn_body,
        grid_spec=grid_spec,
        out_shape=jax.ShapeDtypeStruct((P, DIM), jnp.float32),
        compiler_params=pltpu.CompilerParams(
            dimension_semantics=("arbitrary", "arbitrary")),
    )(te, tv, xs, ws, We1, be1, We2, be2)


@jax.jit
def kernel(x, Wu1, bu1, Wu2, bu2, Wr, We1, be1, We2, be2):
    Wx = Wr[:DIM]
    Wru = Wr[DIM]
    w0, w1, d0, d1, te, tv, aux = _router_call(x, Wu1, bu1, Wu2, bu2, Wx, Wru)

    d0f = d0.reshape(N)
    d1f = d1.reshape(N)
    te = te.reshape(64)[:MAX_TILES]
    tv = tv.reshape(64)[:MAX_TILES]

    w2n = jnp.broadcast_to(jnp.concatenate([w0, w1], axis=0), (2 * N, 128))
    xs, ws = _dispatch_call(x, w2n, d0f, d1f)

    ys = _ffn_call(te, tv, xs, ws, _cast_call(We1),
                   be1.reshape(E, 1, FF), _cast_call(We2),
                   be2.reshape(E, 1, DIM))

    output = _combine_call(ys, d0f, d1f)
    return output, aux.reshape(())


# speedup vs baseline: 1.2124x; 1.2124x over previous
import functools

import jax
import jax.numpy as jnp
from jax.experimental import pallas as pl
from jax.experimental.pallas import tpu as pltpu
from jax.experimental.pallas import tpu_sc as plsc

N = 4096
DIM = 1024
E = 8
K = 2
UD = 16
FF = 4 * DIM

T = 512
MAX_TILES = 24
P = MAX_TILES * T
FT = 4096
JF = FF // FT

_SQRT_2 = 1.4142135623730951


def _erf(x):
    a1, a2, a3, a4, a5 = (
        0.254829592, -0.284496736, 1.421413741, -1.453152027, 1.061405429)
    p = 0.3275911
    s = jnp.sign(x)
    ax = jnp.abs(x)
    t = 1.0 / (1.0 + p * ax)
    poly = ((((a5 * t + a4) * t + a3) * t + a2) * t + a1) * t
    y = 1.0 - poly * jnp.exp(-ax * ax)
    return s * y


def _gelu_exact(x):
    return x * 0.5 * (1.0 + jax.lax.erf(x / _SQRT_2))


def _router_body(x_ref, wu1_ref, bu1_ref, wu2_ref, bu2_ref, wx_ref, wru_ref,
                 w0_ref, w1_ref, d0_ref, d1_ref, te_ref, tv_ref, aux_ref):
    x = x_ref[...]
    h = _gelu_exact(
        jnp.dot(x, wu1_ref[...], preferred_element_type=jnp.float32)
        + bu1_ref[...])
    z = jnp.dot(h, wu2_ref[...], preferred_element_type=jnp.float32) \
        + bu2_ref[...]
    u = jnp.maximum(z, 0.0) + jnp.log(1.0 + jnp.exp(-jnp.abs(z)))
    un = u / (jnp.mean(u) + 1e-8)

    logits = (jnp.dot(x, wx_ref[...], preferred_element_type=jnp.float32)
              + un * wru_ref[...])
    m = jnp.max(logits, axis=-1, keepdims=True)
    ex = jnp.exp(logits - m)
    ssum = jnp.sum(ex, axis=-1, keepdims=True)
    probs = ex / ssum

    neg = jnp.full((N, 1), -1.0, jnp.float32)
    b0 = neg
    b1 = neg
    i0 = jnp.zeros((N, 1), jnp.int32)
    i1 = jnp.zeros((N, 1), jnp.int32)
    for e in range(E):
        pe = probs[:, e:e + 1]
        gt0 = pe > b0
        gt1 = pe > b1
        nb1 = jnp.where(gt0, b0, jnp.where(gt1, pe, b1))
        ni1 = jnp.where(gt0, i0, jnp.where(gt1, e, i1))
        b0 = jnp.where(gt0, pe, b0)
        i0 = jnp.where(gt0, e, i0)
        b1 = nb1
        i1 = ni1
    wsum = b0 + b1
    w0_ref[...] = b0 / wsum
    w1_ref[...] = b1 / wsum

    lanes = jax.lax.broadcasted_iota(jnp.int32, (N, E), 1)
    oh0 = (i0 == lanes).astype(jnp.float32)
    oh1 = (i1 == lanes).astype(jnp.float32)
    c0 = oh0
    c1 = oh1
    s = 1
    while s < N:
        zpad = jnp.zeros((s, E), jnp.float32)
        c0 = c0 + jnp.concatenate([zpad, c0[:N - s]], axis=0)
        c1 = c1 + jnp.concatenate([zpad, c1[:N - s]], axis=0)
        s *= 2
    cnt0 = c0[N - 1:N, :]
    cnt1 = c1[N - 1:N, :]
    cnt = cnt0 + cnt1
    ps = jnp.ceil(cnt / float(T)) * float(T)
    ee = jax.lax.broadcasted_iota(jnp.int32, (E, E), 0)
    jj = jax.lax.broadcasted_iota(jnp.int32, (E, E), 1)
    off = jnp.sum(jnp.where(jj < ee, ps, 0.0), axis=1, keepdims=True)
    off_b = off.reshape(1, E)

    d0 = jnp.sum(oh0 * (off_b + c0 - 1.0), axis=1, keepdims=True)
    d1 = jnp.sum(oh1 * (off_b + cnt0 + c1 - 1.0), axis=1, keepdims=True)
    d0_ref[...] = d0.astype(jnp.int32)
    d1_ref[...] = d1.astype(jnp.int32)

    total = jnp.sum(ps)
    ends = off_b + ps
    tstart = jax.lax.broadcasted_iota(
        jnp.int32, (64, 1), 0).astype(jnp.float32) * float(T)
    te = jnp.sum((jnp.broadcast_to(ends, (64, E)) <= tstart).astype(jnp.int32),
                 axis=1, keepdims=True)
    te_ref[...] = jnp.minimum(te, E - 1)
    tv_ref[...] = (tstart < total).astype(jnp.int32)

    usage = jnp.mean(probs, axis=0, keepdims=True)
    sel = cnt / float(N * K)
    lb = float(E) * jnp.sum(usage * sel)
    lse = m + jnp.log(ssum)
    zl = jnp.mean(lse * lse)
    aux_ref[...] = (0.01 * lb + 0.01 * zl).reshape(1, 1)


def _ffn_body(te_ref, tv_ref, xs_ref, ws_ref, w1_ref, b1_ref, w2_ref, b2_ref,
              ys_ref):
    i = pl.program_id(0)
    j = pl.program_id(1)

    @pl.when(tv_ref[i] == 1)
    def _():
        h = (jnp.dot(xs_ref[...].astype(jnp.bfloat16), w1_ref[0],
                     preferred_element_type=jnp.float32) + b1_ref[0])
        g = _gelu_exact(h)
        contrib = jnp.dot(g.astype(jnp.bfloat16), w2_ref[0],
                          preferred_element_type=jnp.float32)
        if JF == 1:
            ys_ref[...] = (contrib + b2_ref[0]) * ws_ref[:, 0:1]
        else:
            @pl.when(j == 0)
            def _():
                ys_ref[...] = contrib

            @pl.when(jnp.logical_and(j > 0, j < JF - 1))
            def _():
                ys_ref[...] += contrib

            @pl.when(j == JF - 1)
            def _():
                ys_ref[...] = (ys_ref[...] + contrib + b2_ref[0]) \
                    * ws_ref[:, 0:1]


_NC = 2
_NS = 16
_NW = _NC * _NS
_CH = 64
_CH2 = 32


def _dispatch_call(x, w2n, d0, d1):
    mesh = plsc.VectorSubcoreMesh(core_axis_name="c", subcore_axis_name="s")

    @functools.partial(
        pl.kernel, mesh=mesh,
        out_type=(jax.ShapeDtypeStruct((P, DIM), jnp.float32),
                  jax.ShapeDtypeStruct((P, 128), jnp.float32)),
        scratch_types=[
            pltpu.VMEM((_CH, DIM), jnp.float32),
            pltpu.VMEM((_CH, 128), jnp.float32),
            pltpu.VMEM((_CH, 128), jnp.float32),
            pltpu.VMEM((_CH,), jnp.int32),
            pltpu.VMEM((_CH,), jnp.int32),
            pltpu.SemaphoreType.DMA,
            pltpu.SemaphoreType.DMA,
        ])
    def k(x_hbm, w_hbm, d0_hbm, d1_hbm, xs_hbm, ws_hbm,
          rows_v, wa_v, wb_v, i0_v, i1_v, lsem, ssem):
        wid = jax.lax.axis_index("s") * _NC + jax.lax.axis_index("c")
        per_w = N // _NW

        @pl.loop(0, per_w, step=_CH)
        def _(off):
            base = wid * per_w + off
            cps = [
                pltpu.make_async_copy(x_hbm.at[pl.ds(base, _CH)], rows_v,
                                      lsem),
                pltpu.make_async_copy(w_hbm.at[pl.ds(base, _CH)], wa_v, lsem),
                pltpu.make_async_copy(w_hbm.at[pl.ds(N + base, _CH)], wb_v,
                                      lsem),
                pltpu.make_async_copy(d0_hbm.at[pl.ds(base, _CH)], i0_v,
                                      lsem),
                pltpu.make_async_copy(d1_hbm.at[pl.ds(base, _CH)], i1_v,
                                      lsem),
            ]
            for cp in cps:
                cp.start()
            for cp in cps:
                cp.wait()
            scs = [
                pltpu.make_async_copy(rows_v, xs_hbm.at[i0_v], ssem),
                pltpu.make_async_copy(rows_v, xs_hbm.at[i1_v], ssem),
                pltpu.make_async_copy(wa_v, ws_hbm.at[i0_v], ssem),
                pltpu.make_async_copy(wb_v, ws_hbm.at[i1_v], ssem),
            ]
            for cp in scs:
                cp.start()
            for cp in scs:
                cp.wait()

    return k(x, w2n, d0, d1)


def _combine_call(ys, d0, d1):
    mesh = plsc.VectorSubcoreMesh(core_axis_name="c", subcore_axis_name="s")

    @functools.partial(
        pl.kernel, mesh=mesh,
        out_type=jax.ShapeDtypeStruct((N, DIM), jnp.float32),
        scratch_types=[
            pltpu.VMEM((_CH2, DIM), jnp.float32),
            pltpu.VMEM((_CH2, DIM), jnp.float32),
            pltpu.VMEM((_CH2,), jnp.int32),
            pltpu.VMEM((_CH2,), jnp.int32),
            pltpu.SemaphoreType.DMA,
        ])
    def k(ys_hbm, d0_hbm, d1_hbm, o_hbm, g0_v, g1_v, i0_v, i1_v, sem):
        wid = jax.lax.axis_index("s") * _NC + jax.lax.axis_index("c")
        per_w = N // _NW

        @pl.loop(0, per_w, step=_CH2)
        def _(off):
            base = wid * per_w + off
            pltpu.sync_copy(d0_hbm.at[pl.ds(base, _CH2)], i0_v)
            pltpu.sync_copy(d1_hbm.at[pl.ds(base, _CH2)], i1_v)
            cps = [
                pltpu.make_async_copy(ys_hbm.at[i0_v], g0_v, sem),
                pltpu.make_async_copy(ys_hbm.at[i1_v], g1_v, sem),
            ]
            for cp in cps:
                cp.start()
            for cp in cps:
                cp.wait()

            @pl.loop(0, _CH2)
            def _(r):
                @pl.loop(0, DIM, step=64)
                def _(c):
                    for u in range(4):
                        sl = (r, pl.ds(c + 16 * u, 16))
                        g0_v[sl] = g0_v[sl] + g1_v[sl]

            pltpu.sync_copy(g0_v, o_hbm.at[pl.ds(base, _CH2)])

    return k(ys, d0, d1)


def _cast_body(x_ref, o_ref):
    o_ref[...] = x_ref[...].astype(jnp.bfloat16)


def _cast_call(w):
    e, a, b = w.shape
    cb = 1024
    return pl.pallas_call(
        _cast_body,
        grid=(e, b // cb),
        in_specs=[pl.BlockSpec((1, a, cb), lambda i, j: (i, 0, j))],
        out_specs=pl.BlockSpec((1, a, cb), lambda i, j: (i, 0, j)),
        out_shape=jax.ShapeDtypeStruct((e, a, b), jnp.bfloat16),
    )(w)


def _router_call(x, Wu1, bu1, Wu2, bu2, Wx, Wru):
    outs = (
        jax.ShapeDtypeStruct((N, 1), jnp.float32),
        jax.ShapeDtypeStruct((N, 1), jnp.float32),
        jax.ShapeDtypeStruct((N, 1), jnp.int32),
        jax.ShapeDtypeStruct((N, 1), jnp.int32),
        jax.ShapeDtypeStruct((64, 1), jnp.int32),
        jax.ShapeDtypeStruct((64, 1), jnp.int32),
        jax.ShapeDtypeStruct((1, 1), jnp.float32),
    )
    return pl.pallas_call(_router_body, out_shape=outs)(
        x, Wu1, bu1.reshape(1, UD), Wu2, bu2.reshape(1, 1), Wx,
        Wru.reshape(1, E))


def _ffn_call(te, tv, xs, ws, We1, be1, We2, be2):
    grid_spec = pltpu.PrefetchScalarGridSpec(
        num_scalar_prefetch=2,
        grid=(MAX_TILES, JF),
        in_specs=[
            pl.BlockSpec((T, DIM), lambda i, j, te, tv: (i, 0)),
            pl.BlockSpec((T, 128), lambda i, j, te, tv: (i, 0)),
            pl.BlockSpec((1, DIM, FT), lambda i, j, te, tv: (te[i], 0, j)),
            pl.BlockSpec((1, 1, FT), lambda i, j, te, tv: (te[i], 0, j)),
            pl.BlockSpec((1, FT, DIM), lambda i, j, te, tv: (te[i], j, 0)),
            pl.BlockSpec((1, 1, DIM), lambda i, j, te, tv: (te[i], 0, 0)),
        ],
        out_specs=pl.BlockSpec((T, DIM), lambda i, j, te, tv: (i, 0)),
    )
    return pl.pallas_call(
        _ffn_body,
        grid_spec=grid_spec,
        out_shape=jax.ShapeDtypeStruct((P, DIM), jnp.float32),
        compiler_params=pltpu.CompilerParams(
            dimension_semantics=("arbitrary", "arbitrary")),
    )(te, tv, xs, ws, We1, be1, We2, be2)


@jax.jit
def kernel(x, Wu1, bu1, Wu2, bu2, Wr, We1, be1, We2, be2):
    Wx = Wr[:DIM]
    Wru = Wr[DIM]
    w0, w1, d0, d1, te, tv, aux = _router_call(x, Wu1, bu1, Wu2, bu2, Wx, Wru)

    d0f = d0.reshape(N)
    d1f = d1.reshape(N)
    te = te.reshape(64)[:MAX_TILES]
    tv = tv.reshape(64)[:MAX_TILES]

    w2n = jnp.broadcast_to(jnp.concatenate([w0, w1], axis=0), (2 * N, 128))
    xs, ws = _dispatch_call(x, w2n, d0f, d1f)

    ys = _ffn_call(te, tv, xs, ws, _cast_call(We1),
                   be1.reshape(E, 1, FF), _cast_call(We2),
                   be2.reshape(E, 1, DIM))

    output = _combine_call(ys, d0f, d1f)
    return output, aux.reshape(())


# speedup vs baseline: 1.2575x; 1.0372x over previous
import functools

import jax
import jax.numpy as jnp
from jax.experimental import pallas as pl
from jax.experimental.pallas import tpu as pltpu
from jax.experimental.pallas import tpu_sc as plsc

N = 4096
DIM = 1024
E = 8
K = 2
UD = 16
FF = 4 * DIM

T = 512
MAX_TILES = 24
P = MAX_TILES * T
FT = 1024
JF = FF // FT

_SQRT_2 = 1.4142135623730951


def _erf(x):
    a1, a2, a3, a4, a5 = (
        0.254829592, -0.284496736, 1.421413741, -1.453152027, 1.061405429)
    p = 0.3275911
    s = jnp.sign(x)
    ax = jnp.abs(x)
    t = 1.0 / (1.0 + p * ax)
    poly = ((((a5 * t + a4) * t + a3) * t + a2) * t + a1) * t
    y = 1.0 - poly * jnp.exp(-ax * ax)
    return s * y


def _gelu_exact(x):
    return x * 0.5 * (1.0 + jax.lax.erf(x / _SQRT_2))


def _router_body(x_ref, wu1_ref, bu1_ref, wu2_ref, bu2_ref, wx_ref, wru_ref,
                 w0_ref, w1_ref, d0_ref, d1_ref, te_ref, tv_ref, aux_ref):
    x = x_ref[...]
    h = _gelu_exact(
        jnp.dot(x, wu1_ref[...], preferred_element_type=jnp.float32)
        + bu1_ref[...])
    z = jnp.dot(h, wu2_ref[...], preferred_element_type=jnp.float32) \
        + bu2_ref[...]
    u = jnp.maximum(z, 0.0) + jnp.log(1.0 + jnp.exp(-jnp.abs(z)))
    un = u / (jnp.mean(u) + 1e-8)

    logits = (jnp.dot(x, wx_ref[...], preferred_element_type=jnp.float32)
              + un * wru_ref[...])
    m = jnp.max(logits, axis=-1, keepdims=True)
    ex = jnp.exp(logits - m)
    ssum = jnp.sum(ex, axis=-1, keepdims=True)
    probs = ex / ssum

    neg = jnp.full((N, 1), -1.0, jnp.float32)
    b0 = neg
    b1 = neg
    i0 = jnp.zeros((N, 1), jnp.int32)
    i1 = jnp.zeros((N, 1), jnp.int32)
    for e in range(E):
        pe = probs[:, e:e + 1]
        gt0 = pe > b0
        gt1 = pe > b1
        nb1 = jnp.where(gt0, b0, jnp.where(gt1, pe, b1))
        ni1 = jnp.where(gt0, i0, jnp.where(gt1, e, i1))
        b0 = jnp.where(gt0, pe, b0)
        i0 = jnp.where(gt0, e, i0)
        b1 = nb1
        i1 = ni1
    wsum = b0 + b1
    w0_ref[...] = b0 / wsum
    w1_ref[...] = b1 / wsum

    lanes = jax.lax.broadcasted_iota(jnp.int32, (N, E), 1)
    oh0 = (i0 == lanes).astype(jnp.float32)
    oh1 = (i1 == lanes).astype(jnp.float32)
    c0 = oh0
    c1 = oh1
    s = 1
    while s < N:
        zpad = jnp.zeros((s, E), jnp.float32)
        c0 = c0 + jnp.concatenate([zpad, c0[:N - s]], axis=0)
        c1 = c1 + jnp.concatenate([zpad, c1[:N - s]], axis=0)
        s *= 2
    cnt0 = c0[N - 1:N, :]
    cnt1 = c1[N - 1:N, :]
    cnt = cnt0 + cnt1
    ps = jnp.ceil(cnt / float(T)) * float(T)
    ee = jax.lax.broadcasted_iota(jnp.int32, (E, E), 0)
    jj = jax.lax.broadcasted_iota(jnp.int32, (E, E), 1)
    off = jnp.sum(jnp.where(jj < ee, ps, 0.0), axis=1, keepdims=True)
    off_b = off.reshape(1, E)

    d0 = jnp.sum(oh0 * (off_b + c0 - 1.0), axis=1, keepdims=True)
    d1 = jnp.sum(oh1 * (off_b + cnt0 + c1 - 1.0), axis=1, keepdims=True)
    d0_ref[...] = d0.astype(jnp.int32)
    d1_ref[...] = d1.astype(jnp.int32)

    total = jnp.sum(ps)
    ends = off_b + ps
    tstart = jax.lax.broadcasted_iota(
        jnp.int32, (64, 1), 0).astype(jnp.float32) * float(T)
    te = jnp.sum((jnp.broadcast_to(ends, (64, E)) <= tstart).astype(jnp.int32),
                 axis=1, keepdims=True)
    te_ref[...] = jnp.minimum(te, E - 1)
    tv_ref[...] = (tstart < total).astype(jnp.int32)

    usage = jnp.mean(probs, axis=0, keepdims=True)
    sel = cnt / float(N * K)
    lb = float(E) * jnp.sum(usage * sel)
    lse = m + jnp.log(ssum)
    zl = jnp.mean(lse * lse)
    aux_ref[...] = (0.01 * lb + 0.01 * zl).reshape(1, 1)


def _ffn_body(te_ref, tv_ref, xs_ref, ws_ref, w1_ref, b1_ref, w2_ref, b2_ref,
              ys_ref, acc_ref):
    j = pl.program_id(0)
    t = pl.program_id(1)

    @pl.when(tv_ref[t] == 1)
    def _():
        h = (jnp.dot(xs_ref[...].astype(jnp.bfloat16),
                     w1_ref[0].astype(jnp.bfloat16),
                     preferred_element_type=jnp.float32) + b1_ref[0])
        g = _gelu_exact(h)
        c = jnp.dot(g.astype(jnp.bfloat16), w2_ref[0].astype(jnp.bfloat16),
                    preferred_element_type=jnp.float32)
        sl = pl.ds(t * T, T)

        @pl.when(j == 0)
        def _():
            acc_ref[sl, :] = c.astype(jnp.bfloat16)

        @pl.when(jnp.logical_and(j > 0, j < JF - 1))
        def _():
            acc_ref[sl, :] = (acc_ref[sl, :].astype(jnp.float32)
                              + c).astype(jnp.bfloat16)

        @pl.when(j == JF - 1)
        def _():
            ys_ref[...] = (acc_ref[sl, :].astype(jnp.float32) + c
                           + b2_ref[0]) * ws_ref[:, 0:1]


_NC = 2
_NS = 16
_NW = _NC * _NS
_CH = 64
_CH2 = 32


def _dispatch_call(x, w2n, d0, d1):
    mesh = plsc.VectorSubcoreMesh(core_axis_name="c", subcore_axis_name="s")

    @functools.partial(
        pl.kernel, mesh=mesh,
        out_type=(jax.ShapeDtypeStruct((P, DIM), jnp.float32),
                  jax.ShapeDtypeStruct((P, 128), jnp.float32)),
        scratch_types=[
            pltpu.VMEM((_CH, DIM), jnp.float32),
            pltpu.VMEM((_CH, 128), jnp.float32),
            pltpu.VMEM((_CH, 128), jnp.float32),
            pltpu.VMEM((_CH,), jnp.int32),
            pltpu.VMEM((_CH,), jnp.int32),
            pltpu.SemaphoreType.DMA,
            pltpu.SemaphoreType.DMA,
        ])
    def k(x_hbm, w_hbm, d0_hbm, d1_hbm, xs_hbm, ws_hbm,
          rows_v, wa_v, wb_v, i0_v, i1_v, lsem, ssem):
        wid = jax.lax.axis_index("s") * _NC + jax.lax.axis_index("c")
        per_w = N // _NW

        @pl.loop(0, per_w, step=_CH)
        def _(off):
            base = wid * per_w + off
            cps = [
                pltpu.make_async_copy(x_hbm.at[pl.ds(base, _CH)], rows_v,
                                      lsem),
                pltpu.make_async_copy(w_hbm.at[pl.ds(base, _CH)], wa_v, lsem),
                pltpu.make_async_copy(w_hbm.at[pl.ds(N + base, _CH)], wb_v,
                                      lsem),
                pltpu.make_async_copy(d0_hbm.at[pl.ds(base, _CH)], i0_v,
                                      lsem),
                pltpu.make_async_copy(d1_hbm.at[pl.ds(base, _CH)], i1_v,
                                      lsem),
            ]
            for cp in cps:
                cp.start()
            for cp in cps:
                cp.wait()
            scs = [
                pltpu.make_async_copy(rows_v, xs_hbm.at[i0_v], ssem),
                pltpu.make_async_copy(rows_v, xs_hbm.at[i1_v], ssem),
                pltpu.make_async_copy(wa_v, ws_hbm.at[i0_v], ssem),
                pltpu.make_async_copy(wb_v, ws_hbm.at[i1_v], ssem),
            ]
            for cp in scs:
                cp.start()
            for cp in scs:
                cp.wait()

    return k(x, w2n, d0, d1)


def _combine_call(ys, d0, d1):
    mesh = plsc.VectorSubcoreMesh(core_axis_name="c", subcore_axis_name="s")

    @functools.partial(
        pl.kernel, mesh=mesh,
        out_type=jax.ShapeDtypeStruct((N, DIM), jnp.float32),
        scratch_types=[
            pltpu.VMEM((_CH2, DIM), jnp.float32),
            pltpu.VMEM((_CH2, DIM), jnp.float32),
            pltpu.VMEM((_CH2,), jnp.int32),
            pltpu.VMEM((_CH2,), jnp.int32),
            pltpu.SemaphoreType.DMA,
        ])
    def k(ys_hbm, d0_hbm, d1_hbm, o_hbm, g0_v, g1_v, i0_v, i1_v, sem):
        wid = jax.lax.axis_index("s") * _NC + jax.lax.axis_index("c")
        per_w = N // _NW

        @pl.loop(0, per_w, step=_CH2)
        def _(off):
            base = wid * per_w + off
            pltpu.sync_copy(d0_hbm.at[pl.ds(base, _CH2)], i0_v)
            pltpu.sync_copy(d1_hbm.at[pl.ds(base, _CH2)], i1_v)
            cps = [
                pltpu.make_async_copy(ys_hbm.at[i0_v], g0_v, sem),
                pltpu.make_async_copy(ys_hbm.at[i1_v], g1_v, sem),
            ]
            for cp in cps:
                cp.start()
            for cp in cps:
                cp.wait()

            @pl.loop(0, _CH2)
            def _(r):
                @pl.loop(0, DIM, step=64)
                def _(c):
                    for u in range(4):
                        sl = (r, pl.ds(c + 16 * u, 16))
                        g0_v[sl] = g0_v[sl] + g1_v[sl]

            pltpu.sync_copy(g0_v, o_hbm.at[pl.ds(base, _CH2)])

    return k(ys, d0, d1)


def _cast_body(x_ref, o_ref):
    o_ref[...] = x_ref[...].astype(jnp.bfloat16)


def _cast_call(w):
    e, a, b = w.shape
    cb = 1024
    return pl.pallas_call(
        _cast_body,
        grid=(e, b // cb),
        in_specs=[pl.BlockSpec((1, a, cb), lambda i, j: (i, 0, j))],
        out_specs=pl.BlockSpec((1, a, cb), lambda i, j: (i, 0, j)),
        out_shape=jax.ShapeDtypeStruct((e, a, b), jnp.bfloat16),
    )(w)


def _router_call(x, Wu1, bu1, Wu2, bu2, Wx, Wru):
    outs = (
        jax.ShapeDtypeStruct((N, 1), jnp.float32),
        jax.ShapeDtypeStruct((N, 1), jnp.float32),
        jax.ShapeDtypeStruct((N, 1), jnp.int32),
        jax.ShapeDtypeStruct((N, 1), jnp.int32),
        jax.ShapeDtypeStruct((64, 1), jnp.int32),
        jax.ShapeDtypeStruct((64, 1), jnp.int32),
        jax.ShapeDtypeStruct((1, 1), jnp.float32),
    )
    return pl.pallas_call(_router_body, out_shape=outs)(
        x, Wu1, bu1.reshape(1, UD), Wu2, bu2.reshape(1, 1), Wx,
        Wru.reshape(1, E))


def _ffn_call(te, tv, xs, ws, We1, be1, We2, be2):
    grid_spec = pltpu.PrefetchScalarGridSpec(
        num_scalar_prefetch=2,
        grid=(JF, MAX_TILES),
        in_specs=[
            pl.BlockSpec((T, DIM), lambda j, t, te, tv: (t, 0)),
            pl.BlockSpec((T, 128), lambda j, t, te, tv: (t, 0)),
            pl.BlockSpec((1, DIM, FT), lambda j, t, te, tv: (te[t], 0, j)),
            pl.BlockSpec((1, 1, FT), lambda j, t, te, tv: (te[t], 0, j)),
            pl.BlockSpec((1, FT, DIM), lambda j, t, te, tv: (te[t], j, 0)),
            pl.BlockSpec((1, 1, DIM), lambda j, t, te, tv: (te[t], 0, 0)),
        ],
        out_specs=pl.BlockSpec(
            (T, DIM),
            lambda j, t, te, tv: (jnp.where(j == JF - 1, t, MAX_TILES), 0)),
        scratch_shapes=[pltpu.VMEM((P, DIM), jnp.bfloat16)],
    )
    return pl.pallas_call(
        _ffn_body,
        grid_spec=grid_spec,
        out_shape=jax.ShapeDtypeStruct((P + T, DIM), jnp.float32),
        compiler_params=pltpu.CompilerParams(
            dimension_semantics=("arbitrary", "arbitrary")),
    )(te, tv, xs, ws, We1, be1, We2, be2)


@jax.jit
def kernel(x, Wu1, bu1, Wu2, bu2, Wr, We1, be1, We2, be2):
    Wx = Wr[:DIM]
    Wru = Wr[DIM]
    w0, w1, d0, d1, te, tv, aux = _router_call(x, Wu1, bu1, Wu2, bu2, Wx, Wru)

    d0f = d0.reshape(N)
    d1f = d1.reshape(N)
    te = te.reshape(64)[:MAX_TILES]
    tv = tv.reshape(64)[:MAX_TILES]

    w2n = jnp.broadcast_to(jnp.concatenate([w0, w1], axis=0), (2 * N, 128))
    xs, ws = _dispatch_call(x, w2n, d0f, d1f)

    ys = _ffn_call(te, tv, xs, ws, We1, be1.reshape(E, 1, FF),
                   We2, be2.reshape(E, 1, DIM))

    output = _combine_call(ys, d0f, d1f)
    return output, aux.reshape(())
